# Initial kernel scaffold; baseline (speedup 1.0000x reference)
import jax
import jax.numpy as jnp
from jax.experimental import pallas as pl


def kernel(word_seq_id, dep_seq_id, pos_seq_id, target_seq_ids, h0, W_word, W_dep, W_pos, Wih_f, Whh_f, bih_f, bhh_f, Wih_b, Whh_b, bih_b, bhh_b, W_lin, b_lin):
    raise NotImplementedError("write your pallas kernel here")


# speedup vs baseline: 11.9627x; 11.9627x over previous
import jax
import jax.numpy as jnp
from jax import lax
from jax.experimental import pallas as pl
from jax.experimental.pallas import tpu as pltpu

B = 16
L = 2048
HID = 512
IN = 384
T = 128
NBLK = L // T
LOUT = L - 2
NEG = -1e30
F32 = jnp.float32
BF16 = jnp.bfloat16


def _onehot(w, d, p):
    i2 = lax.broadcasted_iota(jnp.int32, (T, B, 512), 2)
    oh = (w[:, :, None] == i2) | (d[:, :, None] == (i2 - 256)) | (p[:, :, None] == (i2 - 384))
    return oh.astype(BF16)


def _proj3(x, wih, bih):
    p0 = jnp.dot(x[:, 0:128], wih[0:128], preferred_element_type=F32)
    p1 = jnp.dot(x[:, 128:256], wih[128:256], preferred_element_type=F32)
    p2 = jnp.dot(x[:, 256:384], wih[256:384], preferred_element_type=F32)
    return ((p0 + p1) + p2) + bih


def _gproj_kernel(wf_ref, df_ref, pf_ref, wb_ref, db_ref, pb_ref,
                  emb_ref, wihf_ref, wihb_ref, bihf_ref, bihb_ref,
                  xwf_ref, xwb_ref):
    emb = emb_ref[...]
    ohf = _onehot(wf_ref[...], df_ref[...], pf_ref[...]).reshape(T * B, 512)
    xf = jnp.dot(ohf, emb, preferred_element_type=F32).astype(BF16)
    xwf_ref[...] = _proj3(xf, wihf_ref[...], bihf_ref[...]).reshape(T, B, HID)
    ohb = _onehot(wb_ref[...], db_ref[...], pb_ref[...]).reshape(T * B, 512)
    xb = jnp.dot(ohb, emb, preferred_element_type=F32).astype(BF16)
    xwb_ref[...] = _proj3(xb, wihb_ref[...], bihb_ref[...]).reshape(T, B, HID)


def _rnn_kernel(xwf_ref, xwb_ref, whhf_ref, whhb_ref, h0f_ref, h0b_ref,
                bhhf_ref, bhhb_ref, outf_ref, outg_ref, hf_s, hb_s):
    i = pl.program_id(0)

    @pl.when(i == 0)
    def _init():
        hf_s[...] = h0f_ref[...]
        hb_s[...] = h0b_ref[...]

    whhf = whhf_ref[...]
    whhb = whhb_ref[...]
    h0b = h0b_ref[...]
    bhhf = bhhf_ref[...]
    bhhb = bhhb_ref[...]
    base_b = (NBLK - 1 - i) * T

    def body(t, carry):
        hf, hb = carry
        accf = jnp.dot(hf, whhf, preferred_element_type=F32)
        hf = jnp.tanh((xwf_ref[t] + accf) + bhhf).astype(BF16)
        outf_ref[t, :, :] = hf
        tb = T - 1 - t
        pg = base_b + tb
        hb_in = jnp.where(pg >= LOUT - 1, h0b, hb)
        accb = jnp.dot(hb_in, whhb, preferred_element_type=F32)
        hb = jnp.tanh((xwb_ref[tb] + accb) + bhhb).astype(BF16)
        outg_ref[tb, :, :] = hb
        return hf, hb

    hf, hb = lax.fori_loop(0, T, body, (hf_s[...], hb_s[...]))
    hf_s[...] = hf
    hb_s[...] = hb


def _cls_kernel(outf_ref, outg_ref, wl1_ref, wl2_ref, bl_ref, lg_ref, pr_ref):
    a = outf_ref[...].astype(F32).reshape(T * B, HID)
    g = outg_ref[...].astype(F32).reshape(T * B, HID)
    lg = (jnp.dot(a, wl1_ref[...], preferred_element_type=F32)
          + jnp.dot(g, wl2_ref[...], preferred_element_type=F32)
          + bl_ref[...])
    m = jnp.max(lg, axis=-1, keepdims=True)
    e = jnp.exp(lg - m)
    pr = e * (1.0 / jnp.sum(e, axis=-1, keepdims=True))
    lg_ref[...] = lg.reshape(T, B, 256)
    pr_ref[...] = pr.reshape(T, B, 256)


def kernel(word_seq_id, dep_seq_id, pos_seq_id, target_seq_ids, h0,
           W_word, W_dep, W_pos,
           Wih_f, Whh_f, bih_f, bhh_f,
           Wih_b, Whh_b, bih_b, bhh_b,
           W_lin, b_lin):
    del target_seq_ids

    wT = word_seq_id.T
    dT = dep_seq_id.T
    pT = pos_seq_id.T

    def shift2(x):
        return jnp.concatenate([x[2:], x[:2]], axis=0)

    wTs, dTs, pTs = shift2(wT), shift2(dT), shift2(pT)

    emb = jnp.zeros((512, IN), BF16)
    emb = emb.at[0:251, 0:128].set(W_word.astype(BF16))
    emb = emb.at[256:307, 128:256].set(W_dep.astype(BF16))
    emb = emb.at[384:435, 256:384].set(W_pos.astype(BF16))

    wihf = Wih_f.T.astype(BF16)
    wihb = Wih_b.T.astype(BF16)
    whhf = Whh_f.T.astype(BF16)
    whhb = Whh_b.T.astype(BF16)
    bihf = bih_f.reshape(1, HID)
    bihb = bih_b.reshape(1, HID)
    bhhf = bhh_f.reshape(1, HID)
    bhhb = bhh_b.reshape(1, HID)
    h0bf = h0.astype(BF16)

    ispec = pl.BlockSpec((T, B), lambda i: (i, 0))
    xspec = pl.BlockSpec((T, B, HID), lambda i: (i, 0, 0))
    inspec = pl.BlockSpec((T, B, IN), lambda i: (i, 0, 0))
    embspec = pl.BlockSpec((512, IN), lambda i: (0, 0))
    wihspec = pl.BlockSpec((IN, HID), lambda i: (0, 0))
    bspec = pl.BlockSpec((1, HID), lambda i: (0, 0))

    xwf, xwb = pl.pallas_call(
        _gproj_kernel,
        grid=(NBLK,),
        in_specs=[ispec] * 6 + [embspec, wihspec, wihspec, bspec, bspec],
        out_specs=[xspec, xspec],
        out_shape=[jax.ShapeDtypeStruct((L, B, HID), F32)] * 2,
        compiler_params=pltpu.CompilerParams(dimension_semantics=("parallel",)),
    )(wT, dT, pT, wTs, dTs, pTs, emb, wihf, wihb, bihf, bihb)

    revh = pl.BlockSpec((T, B, HID), lambda i: (NBLK - 1 - i, 0, 0))
    wspec = pl.BlockSpec((HID, HID), lambda i: (0, 0))
    hspec = pl.BlockSpec((B, HID), lambda i: (0, 0))

    outf, outg = pl.pallas_call(
        _rnn_kernel,
        grid=(NBLK,),
        in_specs=[xspec, revh, wspec, wspec, hspec, hspec, bspec, bspec],
        out_specs=[xspec, revh],
        out_shape=[jax.ShapeDtypeStruct((L, B, HID), BF16)] * 2,
        scratch_shapes=[pltpu.VMEM((B, HID), BF16)] * 2,
        compiler_params=pltpu.CompilerParams(dimension_semantics=("arbitrary",)),
    )(xwf, xwb, whhf, whhb, h0bf[0], h0bf[1], bhhf, bhhb)

    wl1 = jnp.zeros((HID, 256), F32).at[:, :251].set(W_lin[:, :HID].T)
    wl2 = jnp.zeros((HID, 256), F32).at[:, :251].set(W_lin[:, HID:].T)
    bl = jnp.full((1, 256), NEG, F32).at[0, :251].set(b_lin)

    lspec = pl.BlockSpec((T, B, 256), lambda i: (i, 0, 0))
    wlspec = pl.BlockSpec((HID, 256), lambda i: (0, 0))
    blspec = pl.BlockSpec((1, 256), lambda i: (0, 0))

    lg, pr = pl.pallas_call(
        _cls_kernel,
        grid=(NBLK,),
        in_specs=[xspec, xspec, wlspec, wlspec, blspec],
        out_specs=[lspec, lspec],
        out_shape=[jax.ShapeDtypeStruct((L, B, 256), F32)] * 2,
        compiler_params=pltpu.CompilerParams(dimension_semantics=("parallel",)),
    )(outf, outg, wl1, wl2, bl)

    logits = jnp.transpose(lg[:LOUT, :, :251], (1, 0, 2))
    probs = jnp.transpose(pr[:LOUT, :, :251], (1, 0, 2))
    return (logits, probs)
